# baseline (speedup 1.0000x reference)
import jax
import jax.numpy as jnp
from jax import lax
from jax.experimental import pallas as pl
from jax.experimental.pallas import tpu as pltpu

_N = 9216
_D = 64
_K = 1024
_R = 768
_STEPS = _N // _R
_COMMIT = 0.25


def _body(x_ref, w_ref, enc_ref, q_ref, loss_ref, perp_ref,
          counts_ref, sse_ref, w2_ref, iota_ref):
    i = pl.program_id(0)
    x = x_ref[...]
    w = w_ref[...]

    @pl.when(i == 0)
    def _():
        ones_row = jnp.full((1, _D), 1.0, dtype=jnp.float32)
        w2_ref[...] = lax.dot_general(
            ones_row, w * w, (((1,), (1,)), ((), ())),
            preferred_element_type=jnp.float32)
        iota_ref[...] = lax.broadcasted_iota(
            jnp.int32, (_R, _K), 1).astype(jnp.float32)

    x2 = jnp.sum(x * x, axis=1, keepdims=True)
    m2 = lax.dot_general(x * (-2.0), w, (((1,), (1,)), ((), ())),
                         preferred_element_type=jnp.float32)
    dist = (x2 + w2_ref[...]) + m2

    dmin = jnp.min(dist, axis=1, keepdims=True)
    iota = iota_ref[...]
    idx = jnp.min(jnp.where(dist == dmin, iota, float(_K)), axis=1,
                  keepdims=True)

    onehot = (iota == idx).astype(jnp.float32)
    enc_ref[...] = onehot

    q = lax.dot_general(onehot, w, (((1,), (0,)), ((), ())),
                        preferred_element_type=jnp.float32)
    q_ref[...] = x + (q - x)

    part = jnp.sum((q - x) * (q - x))
    csum = jnp.sum(onehot, axis=0, keepdims=True)

    @pl.when(i == 0)
    def _():
        sse_ref[0] = part
        counts_ref[...] = csum

    @pl.when(i > 0)
    def _():
        sse_ref[0] += part
        counts_ref[...] += csum

    @pl.when(i == _STEPS - 1)
    def _():
        mse = sse_ref[0] / float(_N * _D)
        loss_ref[0, 0] = mse + _COMMIT * mse
        avg = counts_ref[...] / float(_N)
        ent = jnp.sum(avg * jnp.log(avg + 1e-10))
        perp_ref[0, 0] = jnp.exp(-ent)


def kernel(inputs, W):
    x = inputs.reshape(_N, _D)
    enc, q, loss, perp = pl.pallas_call(
        _body,
        grid=(_STEPS,),
        in_specs=[
            pl.BlockSpec((_R, _D), lambda i: (i, 0)),
            pl.BlockSpec((_K, _D), lambda i: (0, 0)),
        ],
        out_specs=[
            pl.BlockSpec((_R, _K), lambda i: (i, 0)),
            pl.BlockSpec((_R, _D), lambda i: (i, 0)),
            pl.BlockSpec(memory_space=pltpu.SMEM),
            pl.BlockSpec(memory_space=pltpu.SMEM),
        ],
        out_shape=[
            jax.ShapeDtypeStruct((_N, _K), jnp.float32),
            jax.ShapeDtypeStruct((_N, _D), jnp.float32),
            jax.ShapeDtypeStruct((1, 1), jnp.float32),
            jax.ShapeDtypeStruct((1, 1), jnp.float32),
        ],
        scratch_shapes=[
            pltpu.VMEM((1, _K), jnp.float32),
            pltpu.SMEM((1,), jnp.float32),
            pltpu.VMEM((1, _K), jnp.float32),
            pltpu.VMEM((_R, _K), jnp.float32),
        ],
    )(x, W)
    return (loss[0, 0], q.reshape(inputs.shape), perp[0, 0], enc)


# speedup vs baseline: 1.0382x; 1.0382x over previous
import jax
import jax.numpy as jnp
from jax import lax
from jax.experimental import pallas as pl
from jax.experimental.pallas import tpu as pltpu

_N = 9216
_B = 16
_S = 576
_D = 64
_K = 1024
_BB = 2
_R = _BB * _S
_STEPS = _B // _BB
_COMMIT = 0.25


def _body(x_ref, w_ref, enc_ref, q_ref, loss_ref, perp_ref,
          counts_ref, sse_ref, w2_ref, iota_ref):
    i = pl.program_id(0)
    x = x_ref[...].reshape(_R, _D)
    w = w_ref[...]

    @pl.when(i == 0)
    def _():
        ones_row = jnp.full((1, _D), 1.0, dtype=jnp.float32)
        w2_ref[...] = lax.dot_general(
            ones_row, w * w, (((1,), (1,)), ((), ())),
            preferred_element_type=jnp.float32)
        iota_ref[...] = lax.broadcasted_iota(
            jnp.int32, (_R, _K), 1).astype(jnp.float32)

    x2 = jnp.sum(x * x, axis=1, keepdims=True)
    m2 = lax.dot_general(x * (-2.0), w, (((1,), (1,)), ((), ())),
                         preferred_element_type=jnp.float32)
    dist = (x2 + w2_ref[...]) + m2

    dmin = jnp.min(dist, axis=1, keepdims=True)
    iota = iota_ref[...]
    idx = jnp.min(jnp.where(dist == dmin, iota, float(_K)), axis=1,
                  keepdims=True)

    onehot = (iota == idx).astype(jnp.float32)
    enc_ref[...] = onehot

    q = lax.dot_general(onehot, w, (((1,), (0,)), ((), ())),
                        preferred_element_type=jnp.float32)
    q_ref[...] = (x + (q - x)).reshape(_BB, _S, _D)

    part = jnp.sum((q - x) * (q - x))
    csum = jnp.sum(onehot, axis=0, keepdims=True)

    @pl.when(i == 0)
    def _():
        sse_ref[0] = part
        counts_ref[...] = csum

    @pl.when(i > 0)
    def _():
        sse_ref[0] += part
        counts_ref[...] += csum

    @pl.when(i == _STEPS - 1)
    def _():
        mse = sse_ref[0] / float(_N * _D)
        loss_ref[0, 0] = mse + _COMMIT * mse
        avg = counts_ref[...] / float(_N)
        ent = jnp.sum(avg * jnp.log(avg + 1e-10))
        perp_ref[0, 0] = jnp.exp(-ent)


def kernel(inputs, W):
    enc, q, loss, perp = pl.pallas_call(
        _body,
        grid=(_STEPS,),
        in_specs=[
            pl.BlockSpec((_BB, _S, _D), lambda i: (i, 0, 0)),
            pl.BlockSpec((_K, _D), lambda i: (0, 0)),
        ],
        out_specs=[
            pl.BlockSpec((_R, _K), lambda i: (i, 0)),
            pl.BlockSpec((_BB, _S, _D), lambda i: (i, 0, 0)),
            pl.BlockSpec(memory_space=pltpu.SMEM),
            pl.BlockSpec(memory_space=pltpu.SMEM),
        ],
        out_shape=[
            jax.ShapeDtypeStruct((_N, _K), jnp.float32),
            jax.ShapeDtypeStruct((_B, _S, _D), jnp.float32),
            jax.ShapeDtypeStruct((1, 1), jnp.float32),
            jax.ShapeDtypeStruct((1, 1), jnp.float32),
        ],
        scratch_shapes=[
            pltpu.VMEM((1, _K), jnp.float32),
            pltpu.SMEM((1,), jnp.float32),
            pltpu.VMEM((1, _K), jnp.float32),
            pltpu.VMEM((_R, _K), jnp.float32),
        ],
    )(inputs, W)
    return (loss[0, 0], q, perp[0, 0], enc)


# speedup vs baseline: 1.5120x; 1.4564x over previous
import jax
import jax.numpy as jnp
from jax import lax
from jax.experimental import pallas as pl
from jax.experimental.pallas import tpu as pltpu

_N = 9216
_B = 16
_S = 576
_D = 64
_K = 1024
_COMMIT = 0.25


def _body(xt_ref, wt_ref, enc_ref, qt_ref, loss_ref, perp_ref,
          counts_ref, sse_ref, w2_ref, iota_ref):
    i = pl.program_id(0)
    xt = xt_ref[...].reshape(_D, _S)
    wt = wt_ref[...]

    @pl.when(i == 0)
    def _():
        w2_ref[...] = jnp.sum(wt * wt, axis=0, keepdims=True)
        iota_ref[...] = lax.broadcasted_iota(
            jnp.int32, (_S, _K), 1).astype(jnp.float32)

    x2 = jnp.sum(xt * xt, axis=0, keepdims=True)
    x2c = x2.reshape(_S, 1)
    m2 = lax.dot_general(xt * (-2.0), wt, (((0,), (0,)), ((), ())),
                         preferred_element_type=jnp.float32)
    dist = (x2c + w2_ref[...]) + m2

    dmin = jnp.min(dist, axis=1, keepdims=True)
    iota = iota_ref[...]
    idx = jnp.min(jnp.where(dist == dmin, iota, float(_K)), axis=1,
                  keepdims=True)

    onehot = (iota == idx).astype(jnp.float32)
    enc_ref[...] = onehot

    qt = lax.dot_general(wt, onehot, (((1,), (1,)), ((), ())),
                         preferred_element_type=jnp.float32)
    qt_ref[...] = (xt + (qt - xt)).reshape(1, _D, _S)

    part = jnp.sum((qt - xt) * (qt - xt))
    csum = jnp.sum(onehot, axis=0, keepdims=True)

    @pl.when(i == 0)
    def _():
        sse_ref[0] = part
        counts_ref[...] = csum

    @pl.when(i > 0)
    def _():
        sse_ref[0] += part
        counts_ref[...] += csum

    @pl.when(i == _B - 1)
    def _():
        mse = sse_ref[0] / float(_N * _D)
        loss_ref[0, 0] = mse + _COMMIT * mse
        avg = counts_ref[...] / float(_N)
        ent = jnp.sum(avg * jnp.log(avg + 1e-10))
        perp_ref[0, 0] = jnp.exp(-ent)


def kernel(inputs, W):
    xt = jnp.transpose(inputs, (0, 2, 1))
    wt = W.T
    enc, qt, loss, perp = pl.pallas_call(
        _body,
        grid=(_B,),
        in_specs=[
            pl.BlockSpec((1, _D, _S), lambda i: (i, 0, 0)),
            pl.BlockSpec((_D, _K), lambda i: (0, 0)),
        ],
        out_specs=[
            pl.BlockSpec((_S, _K), lambda i: (i, 0)),
            pl.BlockSpec((1, _D, _S), lambda i: (i, 0, 0)),
            pl.BlockSpec(memory_space=pltpu.SMEM),
            pl.BlockSpec(memory_space=pltpu.SMEM),
        ],
        out_shape=[
            jax.ShapeDtypeStruct((_N, _K), jnp.float32),
            jax.ShapeDtypeStruct((_B, _D, _S), jnp.float32),
            jax.ShapeDtypeStruct((1, 1), jnp.float32),
            jax.ShapeDtypeStruct((1, 1), jnp.float32),
        ],
        scratch_shapes=[
            pltpu.VMEM((1, _K), jnp.float32),
            pltpu.SMEM((1,), jnp.float32),
            pltpu.VMEM((1, _K), jnp.float32),
            pltpu.VMEM((_S, _K), jnp.float32),
        ],
    )(xt, wt)
    q = jnp.transpose(qt, (0, 2, 1))
    return (loss[0, 0], q, perp[0, 0], enc)
